# baseline (speedup 1.0000x reference)
import functools
import math

import jax
import jax.numpy as jnp
from jax import lax
from jax.experimental import pallas as pl
from jax.experimental.pallas import tpu as pltpu
from jax.experimental.pallas import tpu_sc as plsc

D_MODEL = 1024
SCALE = math.sqrt(D_MODEL)
L = 16

NUM_CORES = 2
NUM_SUBCORES = 16
NW = NUM_CORES * NUM_SUBCORES

B_TOTAL = 4 * 8192
B_PER_W = B_TOTAL // NW
CHUNK = 16
N_CHUNKS = B_PER_W // CHUNK
NBUF = 6


def _make_kernel():
    mesh = plsc.VectorSubcoreMesh(
        core_axis_name="c", subcore_axis_name="s",
        num_cores=NUM_CORES, num_subcores=NUM_SUBCORES)

    @functools.partial(
        pl.kernel,
        out_type=jax.ShapeDtypeStruct((B_TOTAL, D_MODEL), jnp.float32),
        mesh=mesh,
        scratch_types=[
            pltpu.VMEM((B_PER_W,), jnp.int32),
            pltpu.VMEM((NBUF, CHUNK, D_MODEL), jnp.float32),
        ] + [pltpu.SemaphoreType.DMA] * (2 * NBUF),
    )
    def emb(x_hbm, table_hbm, out_hbm, idx_v, rows_v, *sems):
        gsems = sems[:NBUF]
        ssems = sems[NBUF:]
        wid = lax.axis_index("s") * NUM_CORES + lax.axis_index("c")
        base = wid * B_PER_W
        pltpu.sync_copy(x_hbm.at[pl.ds(base, B_PER_W)], idx_v)

        def gather(c, p):
            idxs = idx_v.at[pl.ds(c * CHUNK, CHUNK)]
            return pltpu.make_async_copy(table_hbm.at[idxs], rows_v.at[p],
                                         gsems[p])

        def store(c, p):
            return pltpu.make_async_copy(
                rows_v.at[p], out_hbm.at[pl.ds(base + c * CHUNK, CHUNK)],
                ssems[p])

        def scale(p):
            def row_body(r, carry):
                for j in range(D_MODEL // L):
                    v = rows_v[p, r, pl.ds(j * L, L)]
                    rows_v[p, r, pl.ds(j * L, L)] = v * SCALE
                return carry
            lax.fori_loop(0, CHUNK, row_body, 0)

        def steady_step(c, pb, pprev):
            gather(c, pb).wait()
            scale(pb)
            store(c, pb).start()
            store(c - 1, pprev).wait()
            gather(c + NBUF - 1, pprev).start()

        for c in range(NBUF - 1):
            gather(c, c).start()
        gather(0, 0).wait()
        scale(0)
        store(0, 0).start()
        gather(NBUF - 1, NBUF - 1).start()

        n_steady = N_CHUNKS - NBUF
        n_iter = n_steady // NBUF

        def ring_body(co, carry):
            for p in range(NBUF):
                c = 1 + co * NBUF + p
                steady_step(c, (1 + p) % NBUF, p)
            return carry
        lax.fori_loop(0, n_iter, ring_body, 0)

        for c in range(1 + n_iter * NBUF, N_CHUNKS - NBUF + 1):
            steady_step(c, c % NBUF, (c - 1) % NBUF)

        for c in range(N_CHUNKS - NBUF + 1, N_CHUNKS):
            p = c % NBUF
            gather(c, p).wait()
            scale(p)
            store(c, p).start()

        for c in range(N_CHUNKS - NBUF, N_CHUNKS):
            store(c, c % NBUF).wait()

    return emb


_emb = _make_kernel()


def kernel(x, table):
    x_flat = x.reshape(-1).astype(jnp.int32)
    out = _emb(x_flat, table)
    return out.reshape(x.shape + (D_MODEL,))


# speedup vs baseline: 1.0541x; 1.0541x over previous
import functools
import math

import jax
import jax.numpy as jnp
from jax import lax
from jax.experimental import pallas as pl
from jax.experimental.pallas import tpu as pltpu
from jax.experimental.pallas import tpu_sc as plsc

D_MODEL = 1024
SCALE = math.sqrt(D_MODEL)
L = 16

NUM_CORES = 2
NUM_SUBCORES = 16
NW = NUM_CORES * NUM_SUBCORES

B_TOTAL = 4 * 8192
B_PER_W = B_TOTAL // NW
CHUNK = 32
N_CHUNKS = B_PER_W // CHUNK
NBUF = 3


def _make_kernel():
    mesh = plsc.VectorSubcoreMesh(
        core_axis_name="c", subcore_axis_name="s",
        num_cores=NUM_CORES, num_subcores=NUM_SUBCORES)

    @functools.partial(
        pl.kernel,
        out_type=jax.ShapeDtypeStruct((B_TOTAL, D_MODEL), jnp.float32),
        mesh=mesh,
        scratch_types=[
            pltpu.VMEM((B_PER_W,), jnp.int32),
            pltpu.VMEM((NBUF, CHUNK, D_MODEL), jnp.float32),
        ] + [pltpu.SemaphoreType.DMA] * (2 * NBUF),
    )
    def emb(x_hbm, table_hbm, out_hbm, idx_v, rows_v, *sems):
        gsems = sems[:NBUF]
        ssems = sems[NBUF:]
        wid = lax.axis_index("s") * NUM_CORES + lax.axis_index("c")
        base = wid * B_PER_W
        pltpu.sync_copy(x_hbm.at[pl.ds(base, B_PER_W)], idx_v)

        def gather(c, p):
            idxs = idx_v.at[pl.ds(c * CHUNK, CHUNK)]
            return pltpu.make_async_copy(table_hbm.at[idxs], rows_v.at[p],
                                         gsems[p])

        def store(c, p):
            return pltpu.make_async_copy(
                rows_v.at[p], out_hbm.at[pl.ds(base + c * CHUNK, CHUNK)],
                ssems[p])

        def scale(p):
            def row_body(r, carry):
                for j in range(D_MODEL // L):
                    v = rows_v[p, r, pl.ds(j * L, L)]
                    rows_v[p, r, pl.ds(j * L, L)] = v * SCALE
                return carry
            lax.fori_loop(0, CHUNK, row_body, 0)

        def steady_step(c, pb, pprev):
            gather(c, pb).wait()
            scale(pb)
            store(c, pb).start()
            store(c - 1, pprev).wait()
            gather(c + NBUF - 1, pprev).start()

        for c in range(NBUF - 1):
            gather(c, c).start()
        gather(0, 0).wait()
        scale(0)
        store(0, 0).start()
        gather(NBUF - 1, NBUF - 1).start()

        n_steady = N_CHUNKS - NBUF
        n_iter = n_steady // NBUF

        def ring_body(co, carry):
            for p in range(NBUF):
                c = 1 + co * NBUF + p
                steady_step(c, (1 + p) % NBUF, p)
            return carry
        lax.fori_loop(0, n_iter, ring_body, 0)

        for c in range(1 + n_iter * NBUF, N_CHUNKS - NBUF + 1):
            steady_step(c, c % NBUF, (c - 1) % NBUF)

        for c in range(N_CHUNKS - NBUF + 1, N_CHUNKS):
            p = c % NBUF
            gather(c, p).wait()
            scale(p)
            store(c, p).start()

        for c in range(N_CHUNKS - NBUF, N_CHUNKS):
            store(c, c % NBUF).wait()

    return emb


_emb = _make_kernel()


def kernel(x, table):
    x_flat = x.reshape(-1).astype(jnp.int32)
    out = _emb(x_flat, table)
    return out.reshape(x.shape + (D_MODEL,))
